# Initial kernel scaffold; baseline (speedup 1.0000x reference)
import jax
import jax.numpy as jnp
from jax.experimental import pallas as pl


def kernel(user_batch, W1, b1, W2, b2):
    raise NotImplementedError("write your pallas kernel here")


# speedup vs baseline: 5015.2572x; 5015.2572x over previous
import jax
import jax.numpy as jnp
from jax.experimental import pallas as pl


def _gcn_body(x_ref, w1_ref, b1_ref, w2_ref, b2_ref, o_ref):
    x = x_ref[...]
    b, n, _ = x.shape
    d_out = o_ref.shape[-1]
    m = jnp.sum(x, axis=1) * (1.0 / n)
    h1 = jnp.dot(m, w1_ref[...],
                 preferred_element_type=jnp.float32) + b1_ref[...]
    h2 = jnp.dot(h1, w2_ref[...],
                 preferred_element_type=jnp.float32) + b2_ref[...]
    o_ref[...] = jnp.broadcast_to(h2[:, None, :], (b, n, d_out))


def kernel(user_batch, W1, b1, W2, b2):
    B, N, DIN = user_batch.shape
    DHID = W1.shape[1]
    DOUT = W2.shape[1]
    return pl.pallas_call(
        _gcn_body,
        out_shape=jax.ShapeDtypeStruct((B, N, DOUT), user_batch.dtype),
    )(user_batch, W1, b1.reshape(1, DHID), W2, b2.reshape(1, DOUT))
